# baseline (speedup 1.0000x reference)
import functools

import jax
import jax.numpy as jnp
from jax.experimental import pallas as pl
from jax.experimental.pallas import tpu as pltpu

N = 99990
IN_DIM = 128
HID = 32
BLOCK = 16384

_DN_GATE = (((0,), (1,)), ((), ()))
_DN_HEAD = (((1,), (0,)), ((), ()))


def _gclstm_head_kernel(x_ref, wi_ref, wc_ref, wo_ref, bi_ref, bc_ref,
                        bo_ref, wco_ref, lw_ref, lb_ref, out_ref):
    xb = x_ref[...].astype(jnp.bfloat16)
    zi = jax.lax.dot_general(wi_ref[...], xb, _DN_GATE,
                             preferred_element_type=jnp.float32)
    zc = jax.lax.dot_general(wc_ref[...], xb, _DN_GATE,
                             preferred_element_type=jnp.float32)
    zo = jax.lax.dot_general(wo_ref[...], xb, _DN_GATE,
                             preferred_element_type=jnp.float32)
    t1 = jnp.tanh(zi + bi_ref[...])
    t_gate = jnp.tanh(zc + bc_ref[...])
    p = (t1 + 1.0) * t_gate
    t2 = jnp.tanh(zo + bo_ref[...] + wco_ref[...] * p)
    tc = jnp.tanh(0.5 * p)
    h = jax.nn.relu((t2 + 1.0) * tc)
    out_ref[...] = (jax.lax.dot_general(lw_ref[...], h, _DN_HEAD,
                                        preferred_element_type=jnp.float32)
                    + lb_ref[...])


@functools.partial(jax.jit, static_argnames=())
def _run(obs, wi, wc, wo, bi, bc, bo, wco_q, lin_w_half, lin_b):
    grid = (pl.cdiv(N, BLOCK),)
    gate_w = pl.BlockSpec((IN_DIM, HID), lambda i: (0, 0))
    gate_b = pl.BlockSpec((HID, 1), lambda i: (0, 0))
    y = pl.pallas_call(
        _gclstm_head_kernel,
        grid=grid,
        in_specs=[
            pl.BlockSpec((BLOCK, IN_DIM), lambda i: (i, 0)),
            gate_w, gate_w, gate_w,
            gate_b, gate_b, gate_b, gate_b,
            pl.BlockSpec((1, HID), lambda i: (0, 0)),
            pl.BlockSpec((1, 1), lambda i: (0, 0)),
        ],
        out_specs=pl.BlockSpec((1, BLOCK), lambda i: (0, i)),
        out_shape=jax.ShapeDtypeStruct((1, N), jnp.float32),
        compiler_params=pltpu.CompilerParams(
            dimension_semantics=("parallel",)),
    )(obs, wi, wc, wo, bi, bc, bo, wco_q, lin_w_half, lin_b)
    idx = (jnp.arange(90900, dtype=jnp.int32) * 11) // 10 + 1
    return jnp.take(y.reshape(-1), idx)


def kernel(obs, edge_index, W_i, W_f, W_c, W_o, w_c_i, w_c_f, w_c_o, b_i,
           b_f, b_c, b_o, ci_w, ci_b, cf_w, cf_b, cc_w, cc_b, co_w, co_b,
           lin_w, lin_b):
    wi = (0.5 * W_i).astype(jnp.bfloat16)
    wc = W_c.astype(jnp.bfloat16)
    wo = (0.5 * W_o).astype(jnp.bfloat16)
    bi = (0.5 * (b_i + ci_b[None, :])).reshape(HID, 1)
    bc = (b_c + cc_b[None, :]).reshape(HID, 1)
    bo = (0.5 * (b_o + co_b[None, :])).reshape(HID, 1)
    wco_q = (0.25 * w_c_o).reshape(HID, 1)
    return _run(obs, wi, wc, wo, bi, bc, bo, wco_q, (0.5 * lin_w).reshape(1, HID),
                lin_b.reshape(1, 1))


# speedup vs baseline: 7.8668x; 7.8668x over previous
import functools

import jax
import jax.numpy as jnp
from jax.experimental import pallas as pl
from jax.experimental.pallas import tpu as pltpu

N = 99990
IN_DIM = 128
HID = 32
BLOCK = 16384

_DN_GATE = (((0,), (1,)), ((), ()))
_DN_HEAD = (((1,), (0,)), ((), ()))


def _gclstm_head_kernel(x_ref, wi_ref, wc_ref, wo_ref, bi_ref, bc_ref,
                        bo_ref, wco_ref, lw_ref, lb_ref, out_ref):
    xb = x_ref[...].astype(jnp.bfloat16)
    zi = jax.lax.dot_general(wi_ref[...], xb, _DN_GATE,
                             preferred_element_type=jnp.float32)
    zc = jax.lax.dot_general(wc_ref[...], xb, _DN_GATE,
                             preferred_element_type=jnp.float32)
    zo = jax.lax.dot_general(wo_ref[...], xb, _DN_GATE,
                             preferred_element_type=jnp.float32)
    t1 = jnp.tanh(zi + bi_ref[...])
    t_gate = jnp.tanh(zc + bc_ref[...])
    p = (t1 + 1.0) * t_gate
    t2 = jnp.tanh(zo + bo_ref[...] + wco_ref[...] * p)
    tc = jnp.tanh(0.5 * p)
    h = jax.nn.relu((t2 + 1.0) * tc)
    out_ref[...] = (jax.lax.dot_general(lw_ref[...], h, _DN_HEAD,
                                        preferred_element_type=jnp.float32)
                    + lb_ref[...])


@functools.partial(jax.jit, static_argnames=())
def _run(obs, wi, wc, wo, bi, bc, bo, wco_q, lin_w_half, lin_b):
    grid = (pl.cdiv(N, BLOCK),)
    gate_w = pl.BlockSpec((IN_DIM, HID), lambda i: (0, 0))
    gate_b = pl.BlockSpec((HID, 1), lambda i: (0, 0))
    y = pl.pallas_call(
        _gclstm_head_kernel,
        grid=grid,
        in_specs=[
            pl.BlockSpec((BLOCK, IN_DIM), lambda i: (i, 0)),
            gate_w, gate_w, gate_w,
            gate_b, gate_b, gate_b, gate_b,
            pl.BlockSpec((1, HID), lambda i: (0, 0)),
            pl.BlockSpec((1, 1), lambda i: (0, 0)),
        ],
        out_specs=pl.BlockSpec((1, BLOCK), lambda i: (0, i)),
        out_shape=jax.ShapeDtypeStruct((1, N), jnp.float32),
        compiler_params=pltpu.CompilerParams(
            dimension_semantics=("parallel",)),
    )(obs, wi, wc, wo, bi, bc, bo, wco_q, lin_w_half, lin_b)
    sel = jnp.zeros((11, 10), jnp.float32).at[jnp.arange(1, 11), jnp.arange(10)].set(1.0)
    return (y.reshape(9090, 11) @ sel).reshape(-1)


def kernel(obs, edge_index, W_i, W_f, W_c, W_o, w_c_i, w_c_f, w_c_o, b_i,
           b_f, b_c, b_o, ci_w, ci_b, cf_w, cf_b, cc_w, cc_b, co_w, co_b,
           lin_w, lin_b):
    wi = (0.5 * W_i).astype(jnp.bfloat16)
    wc = W_c.astype(jnp.bfloat16)
    wo = (0.5 * W_o).astype(jnp.bfloat16)
    bi = (0.5 * (b_i + ci_b[None, :])).reshape(HID, 1)
    bc = (b_c + cc_b[None, :]).reshape(HID, 1)
    bo = (0.5 * (b_o + co_b[None, :])).reshape(HID, 1)
    wco_q = (0.25 * w_c_o).reshape(HID, 1)
    return _run(obs, wi, wc, wo, bi, bc, bo, wco_q, (0.5 * lin_w).reshape(1, HID),
                lin_b.reshape(1, 1))
